# baseline (speedup 1.0000x reference)
import jax
import jax.numpy as jnp
from jax.experimental import pallas as pl
import jax.experimental.pallas.tpu as pltpu

B, L, H, NH = 4, 1024, 256, 8
DH = H // NH
N = B * L
F32 = jnp.float32
_QSCALE = 1.4426950408889634 / DH ** 0.5
G3 = 3 * H


def _ln(x, g, b, eps=1e-5):
    m = jnp.mean(x, axis=-1, keepdims=True)
    v = jnp.mean((x - m) ** 2, axis=-1, keepdims=True)
    return (x - m) / jnp.sqrt(v + eps) * g + b


def _dot(a, b):
    return jnp.dot(a, b, preferred_element_type=F32)


def _sig(x):
    return 0.5 * jnp.tanh(0.5 * x) + 0.5


def _encode_kernel(aa_ref, ss_ref, dom_ref, prot_ref,
                   aW1, ab1, aW2, ab2, ag, ab,
                   c10, c11, c12, cb1, c20, c21, c22, cb2, sg, sb,
                   dW1, db1, dW2, db2, dg, db,
                   pW1, pb1, pW2, pb2, pg, pb,
                   enc_ref):
    def mlp(x, W1, b1, W2, b2, g, b):
        h = jax.nn.relu(_dot(x, W1) + b1)
        return _ln(_dot(h, W2) + b2, g, b)

    enc_ref[0, 0] = mlp(aa_ref[0], aW1[...], ab1[...], aW2[...], ab2[...],
                        ag[...], ab[...])
    enc_ref[2, 0] = mlp(dom_ref[0], dW1[...], db1[...], dW2[...], db2[...],
                        dg[...], db[...])
    enc_ref[3, 0] = mlp(prot_ref[0], pW1[...], pb1[...], pW2[...], pb2[...],
                        pg[...], pb[...])

    def conv3(x, W0, W1, W2, bb):
        p0 = _dot(x, W0[...])
        p1 = _dot(x, W1[...])
        p2 = _dot(x, W2[...])
        row = jax.lax.broadcasted_iota(jnp.int32, p0.shape, 0)
        down = jnp.where(row == 0, 0.0, pltpu.roll(p0, 1, 0))
        up = jnp.where(row == p0.shape[0] - 1, 0.0,
                       pltpu.roll(p2, p2.shape[0] - 1, 0))
        return down + p1 + up + bb[...]

    h = jax.nn.relu(conv3(ss_ref[0], c10, c11, c12, cb1))
    h = conv3(h, c20, c21, c22, cb2)
    enc_ref[1, 0] = _ln(h, sg[...], sb[...])


def _gnn_kernel(enc_ref, gw, gb2, bu_ref, td_ref):
    def gru(i, x, h):
        Wi = gw[:, (2 * i) * G3:(2 * i + 1) * G3]
        Wh = gw[:, (2 * i + 1) * G3:(2 * i + 2) * G3]
        gi = _dot(x, Wi) + gb2[2 * i]
        gh = _dot(h, Wh) + gb2[2 * i + 1]
        r = _sig(gi[:, :H] + gh[:, :H])
        z = _sig(gi[:, H:2 * H] + gh[:, H:2 * H])
        n = jnp.tanh(gi[:, 2 * H:] + r * gh[:, 2 * H:])
        return (1.0 - z) * n + z * h

    e_aa, e_ss, e_do, e_pr = enc_ref[0], enc_ref[1], enc_ref[2], enc_ref[3]
    bu_ss = gru(0, e_aa, e_ss)
    bu_do = gru(1, bu_ss, e_do)
    bu_pr = gru(2, bu_do, e_pr)
    td_do = gru(3, e_pr, e_do)
    td_ss = gru(4, td_do, e_ss)
    td_aa = gru(5, td_ss, e_aa)
    bu_ref[0] = e_aa
    bu_ref[1] = bu_ss
    bu_ref[2] = bu_do
    bu_ref[3] = bu_pr
    td_ref[0] = td_aa
    td_ref[1] = td_ss
    td_ref[2] = td_do
    td_ref[3] = e_pr


def _integrate_fuse_kernel(bu_ref, td_ref, iw, ib, fw, fb1, fbr, out_ref):
    w = lambda i: iw[:, i * H:(i + 1) * H]
    H2 = 2 * H
    F8 = jnp.float8_e4m3fn

    def integrate(bu, td):
        bg = _sig(_dot(bu, w(0)) + _dot(td, w(1)) + ib[0])
        tg = _sig(_dot(bu, w(2)) + _dot(td, w(3)) + ib[1])
        gb = bu * bg
        gt = td * tg
        q = (_dot(gb, w(4)) + ib[2]).astype(F8)
        k = (_dot(gt, w(5)) + ib[3]).astype(F8)
        v = (_dot(gt, w(6)) + ib[4]).astype(F8)
        outs = []
        for hh in range(NH):
            sl = slice(hh * DH, (hh + 1) * DH)
            qh, kh, vh = q[:, sl], k[:, sl], v[:, sl]
            s = jax.lax.dot_general(qh, kh, (((1,), (1,)), ((), ())),
                                    preferred_element_type=F32)
            e = jnp.exp2(s)
            denom = jnp.sum(e, axis=-1, keepdims=True)
            outs.append(_dot(e.astype(F8), vh) / denom)
        attn = jnp.concatenate(outs, axis=-1)
        ao = _dot(attn, w(7)) + ib[5]
        g2 = gb + gt
        h1 = jax.nn.relu(_dot(ao, w(8)) + _dot(g2, w(9)) + ib[6])
        h2 = _dot(h1, w(10)) + ib[7]
        return _ln(h2, ib[8], ib[9])

    acc = fb1[...]
    for lv in range(4):
        integ = integrate(bu_ref[lv], td_ref[lv])
        acc = acc + _dot(integ, fw[:, lv * H2:(lv + 1) * H2])
    h = jax.nn.relu(acc)
    y = (_dot(h[:, :H], fw[:, 4 * H2:4 * H2 + H])
         + _dot(h[:, H:], fw[:, 4 * H2 + H:4 * H2 + 2 * H]) + fbr[0])
    out_ref[...] = _ln(y, fbr[1], fbr[2])


def _full(shape):
    nd = len(shape)
    return pl.BlockSpec(shape, lambda *args: (0,) * nd)


def _cparams(ndims):
    return pltpu.CompilerParams(dimension_semantics=("arbitrary",) * ndims)


def kernel(aa, ss, domain, protein, params):
    p = params
    r2 = lambda v: v.reshape(1, -1).astype(F32)
    T = lambda w: w.T.astype(F32)

    ea, ed, ep = p['enc_aa'], p['enc_dom'], p['enc_prot']
    es = p['enc_ss']
    cW1 = es['cW1']
    cW2 = es['cW2']
    enc_w = [
        T(ea['W1']), r2(ea['b1']), T(ea['W2']), r2(ea['b2']), r2(ea['ln_g']), r2(ea['ln_b']),
        T(cW1[:, :, 0]), T(cW1[:, :, 1]), T(cW1[:, :, 2]), r2(es['cb1']),
        T(cW2[:, :, 0]), T(cW2[:, :, 1]), T(cW2[:, :, 2]), r2(es['cb2']),
        r2(es['ln_g']), r2(es['ln_b']),
        T(ed['W1']), r2(ed['b1']), T(ed['W2']), r2(ed['b2']), r2(ed['ln_g']), r2(ed['ln_b']),
        T(ep['W1']), r2(ep['b1']), T(ep['W2']), r2(ep['b2']), r2(ep['ln_g']), r2(ep['ln_b']),
    ]
    act_spec = lambda d: pl.BlockSpec((1, L, d), lambda bb: (bb, 0, 0))
    enc = pl.pallas_call(
        _encode_kernel,
        grid=(B,),
        in_specs=[act_spec(64), act_spec(128), act_spec(256), act_spec(512)]
                 + [_full(w.shape) for w in enc_w],
        out_specs=pl.BlockSpec((4, 1, L, H), lambda bb: (0, bb, 0, 0)),
        out_shape=jax.ShapeDtypeStruct((4, B, L, H), F32),
        compiler_params=_cparams(1),
    )(aa.astype(F32), ss.astype(F32), domain.astype(F32), protein.astype(F32), *enc_w)
    enc = enc.reshape(4, N, H)

    gmats, gbias = [], []
    for grp, lv in (('bu_gru', 'ss'), ('bu_gru', 'domain'), ('bu_gru', 'protein'),
                    ('td_gru', 'domain'), ('td_gru', 'ss'), ('td_gru', 'aa')):
        g = p[grp][lv]
        gmats += [T(g['Wih']), T(g['Whh'])]
        gbias += [r2(g['bih']), r2(g['bhh'])]
    gw = jnp.concatenate(gmats, axis=1)
    gb2 = jnp.concatenate(gbias, axis=0)
    BT = 512
    bu, td = pl.pallas_call(
        _gnn_kernel,
        grid=(N // BT,),
        in_specs=[pl.BlockSpec((4, BT, H), lambda t: (0, t, 0)),
                  _full(gw.shape), _full(gb2.shape)],
        out_specs=[pl.BlockSpec((4, BT, H), lambda t: (0, t, 0))] * 2,
        out_shape=[jax.ShapeDtypeStruct((4, N, H), F32)] * 2,
        compiler_params=_cparams(1),
    )(enc, gw, gb2)

    it = p['integ']
    Win, bin_ = it['in_proj_w'], it['in_proj_b']
    iw = jnp.concatenate([
        T(it['bug_W'][:, :H]), T(it['bug_W'][:, H:]),
        T(it['tdg_W'][:, :H]), T(it['tdg_W'][:, H:]),
        T(Win[:H]) * _QSCALE, T(Win[H:2 * H]), T(Win[2 * H:]),
        T(it['out_proj_w']),
        T(it['fi_W1'][:, :H]), T(it['fi_W1'][:, H:]), T(it['fi_W2']),
    ], axis=1)
    ib = jnp.concatenate([
        r2(it['bug_b']), r2(it['tdg_b']),
        r2(bin_[:H]) * _QSCALE, r2(bin_[H:2 * H]), r2(bin_[2 * H:]),
        r2(it['out_proj_b']), r2(it['fi_b1']), r2(it['fi_b2']),
        r2(it['fi_ln_g']), r2(it['fi_ln_b']),
    ], axis=0)
    fu = p['fus']
    W1t = T(fu['W1'])
    W2t = T(fu['W2'])
    fw = jnp.concatenate([W1t[:H], W1t[H:2 * H], W1t[2 * H:3 * H], W1t[3 * H:],
                          W2t[:H], W2t[H:]], axis=1)
    fbr = jnp.concatenate([r2(fu['b2']), r2(fu['ln_g']), r2(fu['ln_b'])], axis=0)
    lvl_spec = pl.BlockSpec((4, L, H), lambda bb: (0, bb, 0))
    out = pl.pallas_call(
        _integrate_fuse_kernel,
        grid=(B,),
        in_specs=[lvl_spec, lvl_spec, _full(iw.shape), _full(ib.shape),
                  _full(fw.shape), _full((1, 2 * H)), _full(fbr.shape)],
        out_specs=pl.BlockSpec((L, H), lambda bb: (bb, 0)),
        out_shape=jax.ShapeDtypeStruct((N, H), F32),
        compiler_params=_cparams(1),
    )(bu, td, iw, ib, fw, r2(fu['b1']), fbr)
    return out.reshape(B, L, H)


# speedup vs baseline: 1.4744x; 1.4744x over previous
import jax
import jax.numpy as jnp
from jax.experimental import pallas as pl
import jax.experimental.pallas.tpu as pltpu

B, L, H, NH = 4, 1024, 256, 8
DH = H // NH
N = B * L
F32 = jnp.float32
_QSCALE = 1.4426950408889634 / DH ** 0.5
G3 = 3 * H


def _ln(x, g, b, eps=1e-5):
    m = jnp.mean(x, axis=-1, keepdims=True)
    v = jnp.mean((x - m) ** 2, axis=-1, keepdims=True)
    return (x - m) / jnp.sqrt(v + eps) * g + b


def _dot(a, b):
    return jnp.dot(a, b, preferred_element_type=F32)


def _sig(x):
    return 0.5 * jnp.tanh(0.5 * x) + 0.5


def _encode_kernel(aa_ref, ss_ref, dom_ref, prot_ref,
                   aW1, ab1, aW2, ab2, ag, ab,
                   c10, c11, c12, cb1, c20, c21, c22, cb2, sg, sb,
                   dW1, db1, dW2, db2, dg, db,
                   pW1, pb1, pW2, pb2, pg, pb,
                   enc_ref):
    def mlp(x, W1, b1, W2, b2, g, b):
        h = jax.nn.relu(_dot(x, W1) + b1)
        return _ln(_dot(h, W2) + b2, g, b)

    enc_ref[0, 0] = mlp(aa_ref[0], aW1[...], ab1[...], aW2[...], ab2[...],
                        ag[...], ab[...])
    enc_ref[2, 0] = mlp(dom_ref[0], dW1[...], db1[...], dW2[...], db2[...],
                        dg[...], db[...])
    enc_ref[3, 0] = mlp(prot_ref[0], pW1[...], pb1[...], pW2[...], pb2[...],
                        pg[...], pb[...])

    def conv3(x, W0, W1, W2, bb):
        p0 = _dot(x, W0[...])
        p1 = _dot(x, W1[...])
        p2 = _dot(x, W2[...])
        row = jax.lax.broadcasted_iota(jnp.int32, p0.shape, 0)
        down = jnp.where(row == 0, 0.0, pltpu.roll(p0, 1, 0))
        up = jnp.where(row == p0.shape[0] - 1, 0.0,
                       pltpu.roll(p2, p2.shape[0] - 1, 0))
        return down + p1 + up + bb[...]

    h = jax.nn.relu(conv3(ss_ref[0], c10, c11, c12, cb1))
    h = conv3(h, c20, c21, c22, cb2)
    enc_ref[1, 0] = _ln(h, sg[...], sb[...])


def _gnn_kernel(enc_ref, gw, gb2, bu_ref, td_ref):
    def gru(i, x, h):
        Wi = gw[:, (2 * i) * G3:(2 * i + 1) * G3]
        Wh = gw[:, (2 * i + 1) * G3:(2 * i + 2) * G3]
        gi = _dot(x, Wi) + gb2[2 * i]
        gh = _dot(h, Wh) + gb2[2 * i + 1]
        r = _sig(gi[:, :H] + gh[:, :H])
        z = _sig(gi[:, H:2 * H] + gh[:, H:2 * H])
        n = jnp.tanh(gi[:, 2 * H:] + r * gh[:, 2 * H:])
        return (1.0 - z) * n + z * h

    e_aa, e_ss, e_do, e_pr = enc_ref[0], enc_ref[1], enc_ref[2], enc_ref[3]
    bu_ss = gru(0, e_aa, e_ss)
    bu_do = gru(1, bu_ss, e_do)
    bu_pr = gru(2, bu_do, e_pr)
    td_do = gru(3, e_pr, e_do)
    td_ss = gru(4, td_do, e_ss)
    td_aa = gru(5, td_ss, e_aa)
    bu_ref[0] = e_aa
    bu_ref[1] = bu_ss
    bu_ref[2] = bu_do
    bu_ref[3] = bu_pr
    td_ref[0] = td_aa
    td_ref[1] = td_ss
    td_ref[2] = td_do
    td_ref[3] = e_pr


def _integrate_kernel(bu_ref, td_ref, iw, ib, out_ref):
    w = lambda i: iw[:, i * H:(i + 1) * H]
    F8 = jnp.float8_e4m3fn
    bu = bu_ref[0]
    td = td_ref[0]
    bg = _sig(_dot(bu, w(0)) + _dot(td, w(1)) + ib[0])
    tg = _sig(_dot(bu, w(2)) + _dot(td, w(3)) + ib[1])
    gb = bu * bg
    gt = td * tg
    q = (_dot(gb, w(4)) + ib[2]).astype(F8)
    k = (_dot(gt, w(5)) + ib[3]).astype(F8)
    v = (_dot(gt, w(6)) + ib[4]).astype(F8)
    outs = []
    for hh in range(NH):
        sl = slice(hh * DH, (hh + 1) * DH)
        qh, kh, vh = q[:, sl], k[:, sl], v[:, sl]
        s = jax.lax.dot_general(qh, kh, (((1,), (1,)), ((), ())),
                                preferred_element_type=F32)
        e = jnp.exp2(s)
        denom = jnp.sum(e, axis=-1, keepdims=True)
        outs.append(_dot(e.astype(F8), vh) / denom)
    attn = jnp.concatenate(outs, axis=-1)
    ao = _dot(attn, w(7)) + ib[5]
    g2 = gb + gt
    h1 = jax.nn.relu(_dot(ao, w(8)) + _dot(g2, w(9)) + ib[6])
    h2 = _dot(h1, w(10)) + ib[7]
    out_ref[0] = _ln(h2, ib[8], ib[9])


def _fuse_kernel(integ_ref, fw, fb1, fbr, out_ref):
    H2 = 2 * H
    h = jax.nn.relu(_dot(integ_ref[0], fw[:, 0:H2])
                    + _dot(integ_ref[1], fw[:, H2:2 * H2])
                    + _dot(integ_ref[2], fw[:, 2 * H2:3 * H2])
                    + _dot(integ_ref[3], fw[:, 3 * H2:4 * H2])
                    + fb1[...])
    y = (_dot(h[:, :H], fw[:, 4 * H2:4 * H2 + H])
         + _dot(h[:, H:], fw[:, 4 * H2 + H:4 * H2 + 2 * H]) + fbr[0])
    out_ref[...] = _ln(y, fbr[1], fbr[2])


def _full(shape):
    nd = len(shape)
    return pl.BlockSpec(shape, lambda *args: (0,) * nd)


def _cparams(ndims):
    return pltpu.CompilerParams(dimension_semantics=("arbitrary",) * ndims)


def kernel(aa, ss, domain, protein, params):
    p = params
    r2 = lambda v: v.reshape(1, -1).astype(F32)
    T = lambda w: w.T.astype(F32)

    ea, ed, ep = p['enc_aa'], p['enc_dom'], p['enc_prot']
    es = p['enc_ss']
    cW1 = es['cW1']
    cW2 = es['cW2']
    enc_w = [
        T(ea['W1']), r2(ea['b1']), T(ea['W2']), r2(ea['b2']), r2(ea['ln_g']), r2(ea['ln_b']),
        T(cW1[:, :, 0]), T(cW1[:, :, 1]), T(cW1[:, :, 2]), r2(es['cb1']),
        T(cW2[:, :, 0]), T(cW2[:, :, 1]), T(cW2[:, :, 2]), r2(es['cb2']),
        r2(es['ln_g']), r2(es['ln_b']),
        T(ed['W1']), r2(ed['b1']), T(ed['W2']), r2(ed['b2']), r2(ed['ln_g']), r2(ed['ln_b']),
        T(ep['W1']), r2(ep['b1']), T(ep['W2']), r2(ep['b2']), r2(ep['ln_g']), r2(ep['ln_b']),
    ]
    act_spec = lambda d: pl.BlockSpec((1, L, d), lambda bb: (bb, 0, 0))
    enc = pl.pallas_call(
        _encode_kernel,
        grid=(B,),
        in_specs=[act_spec(64), act_spec(128), act_spec(256), act_spec(512)]
                 + [_full(w.shape) for w in enc_w],
        out_specs=pl.BlockSpec((4, 1, L, H), lambda bb: (0, bb, 0, 0)),
        out_shape=jax.ShapeDtypeStruct((4, B, L, H), F32),
        compiler_params=_cparams(1),
    )(aa.astype(F32), ss.astype(F32), domain.astype(F32), protein.astype(F32), *enc_w)
    enc = enc.reshape(4, N, H)

    gmats, gbias = [], []
    for grp, lv in (('bu_gru', 'ss'), ('bu_gru', 'domain'), ('bu_gru', 'protein'),
                    ('td_gru', 'domain'), ('td_gru', 'ss'), ('td_gru', 'aa')):
        g = p[grp][lv]
        gmats += [T(g['Wih']), T(g['Whh'])]
        gbias += [r2(g['bih']), r2(g['bhh'])]
    gw = jnp.concatenate(gmats, axis=1)
    gb2 = jnp.concatenate(gbias, axis=0)
    BT = 512
    bu, td = pl.pallas_call(
        _gnn_kernel,
        grid=(N // BT,),
        in_specs=[pl.BlockSpec((4, BT, H), lambda t: (0, t, 0)),
                  _full(gw.shape), _full(gb2.shape)],
        out_specs=[pl.BlockSpec((4, BT, H), lambda t: (0, t, 0))] * 2,
        out_shape=[jax.ShapeDtypeStruct((4, N, H), F32)] * 2,
        compiler_params=_cparams(1),
    )(enc, gw, gb2)

    it = p['integ']
    Win, bin_ = it['in_proj_w'], it['in_proj_b']
    iw = jnp.concatenate([
        T(it['bug_W'][:, :H]), T(it['bug_W'][:, H:]),
        T(it['tdg_W'][:, :H]), T(it['tdg_W'][:, H:]),
        T(Win[:H]) * _QSCALE, T(Win[H:2 * H]), T(Win[2 * H:]),
        T(it['out_proj_w']),
        T(it['fi_W1'][:, :H]), T(it['fi_W1'][:, H:]), T(it['fi_W2']),
    ], axis=1)
    ib = jnp.concatenate([
        r2(it['bug_b']), r2(it['tdg_b']),
        r2(bin_[:H]) * _QSCALE, r2(bin_[H:2 * H]), r2(bin_[2 * H:]),
        r2(it['out_proj_b']), r2(it['fi_b1']), r2(it['fi_b2']),
        r2(it['fi_ln_g']), r2(it['fi_ln_b']),
    ], axis=0)
    lvl_spec = pl.BlockSpec((1, L, H), lambda lv, bb: (lv, bb, 0))
    integ = pl.pallas_call(
        _integrate_kernel,
        grid=(4, B),
        in_specs=[lvl_spec, lvl_spec, _full(iw.shape), _full(ib.shape)],
        out_specs=pl.BlockSpec((1, L, H), lambda lv, bb: (lv, bb, 0)),
        out_shape=jax.ShapeDtypeStruct((4, N, H), F32),
        compiler_params=_cparams(2),
    )(bu, td, iw, ib)

    fu = p['fus']
    W1t = T(fu['W1'])
    W2t = T(fu['W2'])
    fw = jnp.concatenate([W1t[:H], W1t[H:2 * H], W1t[2 * H:3 * H], W1t[3 * H:],
                          W2t[:H], W2t[H:]], axis=1)
    fbr = jnp.concatenate([r2(fu['b2']), r2(fu['ln_g']), r2(fu['ln_b'])], axis=0)
    FT = 1024
    out = pl.pallas_call(
        _fuse_kernel,
        grid=(N // FT,),
        in_specs=[pl.BlockSpec((4, FT, H), lambda t: (0, t, 0)),
                  _full(fw.shape), _full((1, 2 * H)), _full(fbr.shape)],
        out_specs=pl.BlockSpec((FT, H), lambda t: (t, 0)),
        out_shape=jax.ShapeDtypeStruct((N, H), F32),
        compiler_params=_cparams(1),
    )(integ, fw, r2(fu['b1']), fbr)
    return out.reshape(B, L, H)
